# baseline (speedup 1.0000x reference)
import functools

import jax
import jax.numpy as jnp
from jax import lax
from jax.experimental import pallas as pl
from jax.experimental.pallas import tpu as pltpu
from jax.experimental.pallas import tpu_sc as plsc

_NC = 19
_SMOOTH = 0.01
_DELTA = 0.1

_L = 16
_TR = 8
_TC = 256
_T = _TR * _TC
_BINS = _NC * _L
_RPW = 48
_TCBH = 32


def _sc_body(rows_per_w, x_hbm, lab_hbm, t_hbm, parts_hbm,
             t_v, lab_v, x_v, acc_s, acc_c, sem_x0, sem_x1, sem_l0,
             sem_l1):
    ncores = 2
    wid = lax.axis_index("s") * ncores + lax.axis_index("c")
    H = x_hbm.shape[2]
    W = x_hbm.shape[3]
    grow0 = wid * rows_per_w

    cpr = W // _TC
    n_tiles = (rows_per_w // _TR) * cpr
    sems_x = (sem_x0, sem_x1)
    sems_l = (sem_l0, sem_l1)

    def tile_copies(t, buf):
        grow = grow0 + (t // cpr) * _TR
        b = grow // H
        r_off = grow % H
        c_off = (t % cpr) * _TC
        cx = pltpu.make_async_copy(
            x_hbm.at[b, :, pl.ds(r_off, _TR), pl.ds(c_off, _TC)],
            x_v.at[buf], sems_x[buf])
        cl = pltpu.make_async_copy(
            lab_hbm.at[b, pl.ds(r_off, _TR), pl.ds(c_off, _TC)],
            lab_v.at[buf], sems_l[buf])
        return cx, cl

    def start_tile(t, buf):
        for c in tile_copies(t, buf):
            c.start()

    def wait_tile(t, buf):
        for c in tile_copies(t, buf):
            c.wait()

    start_tile(0, 0)

    pltpu.sync_copy(t_hbm, t_v)

    zero16 = jnp.zeros((_L,), jnp.float32)
    for i in range(_NC):
        acc_s[pl.ds(i * _L, _L)] = zero16
        acc_c[pl.ds(i * _L, _L)] = zero16

    iota19 = lax.iota(jnp.int32, _L) * _NC
    ones16 = jnp.ones((_L,), jnp.float32)

    for t in range(n_tiles):
        buf = t % 2
        if t + 1 < n_tiles:
            start_tile(t + 1, (t + 1) % 2)
        wait_tile(t, buf)

        def vec_body(v, carry, buf=buf):
            r = lax.shift_right_logical(v, 4)
            base = (v & 15) * _L
            lab16 = lab_v[buf, r, pl.ds(base, _L)]
            y = zero16
            for c in range(_NC):
                x = x_v[buf, c, r, pl.ds(base, _L)]
                g = plsc.load_gather(t_v.at[pl.ds(c * 24, 24)], [lab16])
                a = plsc.bitcast(lax.shift_left(g, 16), jnp.float32)
                s = plsc.bitcast(g & jnp.int32(-65536), jnp.float32)
                y = y + jnp.maximum(jnp.abs(x - a) * s, _DELTA)
            sidx = lab16 + iota19
            plsc.addupdate_scatter(acc_s, [sidx], y)
            plsc.addupdate_scatter(acc_c, [sidx], ones16)
            return carry

        lax.fori_loop(0, _T // _L, vec_body, 0)

    pltpu.sync_copy(acc_s, parts_hbm.at[wid, 0])
    pltpu.sync_copy(acc_c, parts_hbm.at[wid, 1])


def _tc_tile_body(x_ref, g_ref, a_ref, s_ref, sums_ref, cnts_ref):
    @pl.when(pl.program_id(0) == 0)
    def _init():
        sums_ref[...] = jnp.zeros_like(sums_ref)
        cnts_ref[...] = jnp.zeros_like(cnts_ref)

    g = g_ref[0]
    for l in range(1, _NC):
        mask = (g == l).astype(jnp.float32)
        acc = None
        for c in range(_NC):
            t = jnp.maximum(jnp.abs(x_ref[0, c] - a_ref[l, c]) * s_ref[l, c],
                            _DELTA)
            acc = t if acc is None else acc + t
        masked = acc * mask
        sums_ref[l, :] += jnp.sum(masked, axis=0)
        cnts_ref[l, :] += jnp.sum(mask, axis=0)


def _owloss_tc_part(logits, sem_gt, a_tab, s_tab, b_img, row0):
    B, C, H, W = logits.shape
    nh = (H - row0) // _TCBH
    h0 = row0 // _TCBH
    out = pl.pallas_call(
        _tc_tile_body,
        grid=(nh,),
        in_specs=[
            pl.BlockSpec((1, C, _TCBH, W), lambda i: (b_img, 0, h0 + i, 0)),
            pl.BlockSpec((1, _TCBH, W), lambda i: (b_img, h0 + i, 0)),
            pl.BlockSpec(memory_space=pltpu.SMEM),
            pl.BlockSpec(memory_space=pltpu.SMEM),
        ],
        out_specs=[
            pl.BlockSpec((24, W), lambda i: (0, 0)),
            pl.BlockSpec((24, W), lambda i: (0, 0)),
        ],
        out_shape=[
            jax.ShapeDtypeStruct((24, W), jnp.float32),
            jax.ShapeDtypeStruct((24, W), jnp.float32),
        ],
    )(logits, sem_gt, a_tab, s_tab)
    return out


@jax.jit
def _owloss_sc(x4, lab, t_tab):
    mesh = plsc.VectorSubcoreMesh(core_axis_name="c", subcore_axis_name="s")
    nw = mesh.num_cores * mesh.num_subcores
    B, C, H, W = x4.shape
    body = functools.partial(_sc_body, _RPW)
    parts = pl.kernel(
        body,
        out_type=jax.ShapeDtypeStruct((nw, 2, _BINS), jnp.float32),
        mesh=mesh,
        compiler_params=pltpu.CompilerParams(needs_layout_passes=False),
        scratch_types=[
            pltpu.VMEM((t_tab.shape[0],), jnp.int32),
            pltpu.VMEM((2, _TR, _TC), jnp.int32),
            pltpu.VMEM((2, _NC, _TR, _TC), jnp.float32),
            pltpu.VMEM((_BINS,), jnp.float32),
            pltpu.VMEM((_BINS,), jnp.float32),
            pltpu.SemaphoreType.DMA,
            pltpu.SemaphoreType.DMA,
            pltpu.SemaphoreType.DMA,
            pltpu.SemaphoreType.DMA,
        ],
    )(x4, lab, t_tab)
    return parts


def kernel(logits, sem_gt, is_train, previous_features, previous_count, var):
    B, C, H, W = logits.shape
    pos = var > 0
    absv = jnp.abs(var)
    nzmin = jnp.min(jnp.where(pos, absv, jnp.inf), axis=1, keepdims=True)
    variance = jnp.where(pos, nzmin, var)
    norm_var = variance / nzmin
    s_full = 1.0 / (norm_var + _SMOOTH)

    a_u = lax.bitcast_convert_type(
        previous_features.T.astype(jnp.bfloat16), jnp.uint16).astype(jnp.uint32)
    s_u = lax.bitcast_convert_type(
        s_full.T.astype(jnp.bfloat16), jnp.uint16).astype(jnp.uint32)
    word = ((s_u << 16) | a_u).astype(jnp.uint32)
    t_tab = jnp.zeros((_NC, 24), jnp.uint32).at[:, :_NC].set(word)
    t_tab = lax.bitcast_convert_type(t_tab.reshape(-1), jnp.int32)

    lab = sem_gt.astype(jnp.int32)

    parts = _owloss_sc(logits, lab, t_tab)
    tc_sums2d, tc_cnts2d = _owloss_tc_part(
        logits, lab, previous_features, s_full, B - 1, 0)
    tc_sums = jnp.sum(tc_sums2d[: _NC], axis=1)
    tc_cnts = jnp.sum(tc_cnts2d[: _NC], axis=1)
    sums = parts[:, 0, :].reshape(-1, _L, _NC).sum(axis=(0, 1))
    cnts = parts[:, 1, :].reshape(-1, _L, _NC).sum(axis=(0, 1))
    sums = sums + tc_sums
    cnts = cnts + tc_cnts
    sums = sums - (_NC * _DELTA) * cnts

    lbl = jnp.arange(_NC)
    denom = jnp.maximum(cnts * _NC, 1.0)
    mean_val = sums / denom
    cond = (lbl >= 1) & (cnts > 0) & (previous_count > 0) & (jnp.sum(var, axis=1) != 0)
    acc = jnp.sum(jnp.where(cond, mean_val, 0.0))
    return jnp.clip(acc, 0.0, 20.0)


# speedup vs baseline: 1.0068x; 1.0068x over previous
import functools

import jax
import jax.numpy as jnp
from jax import lax
from jax.experimental import pallas as pl
from jax.experimental.pallas import tpu as pltpu
from jax.experimental.pallas import tpu_sc as plsc

_NC = 19
_SMOOTH = 0.01
_DELTA = 0.1

_L = 16
_TR = 8
_TC = 256
_T = _TR * _TC
_BINS = _NC * _L
_RPW = 48
_TCBH = 32


def _sc_body(rows_per_w, x_hbm, lab_hbm, t_hbm, parts_hbm,
             t_v, lab_v, x_v, acc_s, acc_c, sem_x0, sem_x1, sem_l0,
             sem_l1):
    ncores = 2
    wid = lax.axis_index("s") * ncores + lax.axis_index("c")
    H = x_hbm.shape[2]
    W = x_hbm.shape[3]
    grow0 = wid * rows_per_w

    cpr = W // _TC
    n_tiles = (rows_per_w // _TR) * cpr
    sems_x = (sem_x0, sem_x1)
    sems_l = (sem_l0, sem_l1)

    def tile_copies(t, buf):
        grow = grow0 + (t // cpr) * _TR
        b = grow // H
        r_off = grow % H
        c_off = (t % cpr) * _TC
        cx = pltpu.make_async_copy(
            x_hbm.at[b, :, pl.ds(r_off, _TR), pl.ds(c_off, _TC)],
            x_v.at[buf], sems_x[buf])
        cl = pltpu.make_async_copy(
            lab_hbm.at[b, pl.ds(r_off, _TR), pl.ds(c_off, _TC)],
            lab_v.at[buf], sems_l[buf])
        return cx, cl

    def start_tile(t, buf):
        for c in tile_copies(t, buf):
            c.start()

    def wait_tile(t, buf):
        for c in tile_copies(t, buf):
            c.wait()

    start_tile(0, 0)

    pltpu.sync_copy(t_hbm, t_v)

    zero16 = jnp.zeros((_L,), jnp.float32)
    for i in range(_NC):
        acc_s[pl.ds(i * _L, _L)] = zero16
        acc_c[pl.ds(i * _L, _L)] = zero16

    iota16 = lax.iota(jnp.int32, _L)
    ones16 = jnp.ones((_L,), jnp.float32)

    for t in range(n_tiles):
        buf = t % 2
        if t + 1 < n_tiles:
            start_tile(t + 1, (t + 1) % 2)
        wait_tile(t, buf)

        def vec_body(v, carry, buf=buf):
            r = lax.shift_right_logical(v, 4)
            base = (v & 15) * _L
            lab16 = lab_v[buf, r, pl.ds(base, _L)]
            y = zero16
            for c in range(_NC):
                x = x_v[buf, c, r, pl.ds(base, _L)]
                g = plsc.load_gather(t_v.at[pl.ds(c * 24, 24)], [lab16])
                a = plsc.bitcast(lax.shift_left(g, 16), jnp.float32)
                s = plsc.bitcast(g & jnp.int32(-65536), jnp.float32)
                y = y + jnp.maximum(jnp.abs(x - a) * s, _DELTA)
            sidx = lab16 * _L + iota16
            plsc.addupdate_scatter(acc_s, [sidx], y)
            plsc.addupdate_scatter(acc_c, [sidx], ones16)
            return carry

        lax.fori_loop(0, _T // _L, vec_body, 0)

    pltpu.sync_copy(acc_s, parts_hbm.at[wid, 0])
    pltpu.sync_copy(acc_c, parts_hbm.at[wid, 1])


def _tc_tile_body(x_ref, g_ref, a_ref, s_ref, sums_ref, cnts_ref):
    @pl.when(pl.program_id(0) == 0)
    def _init():
        sums_ref[...] = jnp.zeros_like(sums_ref)
        cnts_ref[...] = jnp.zeros_like(cnts_ref)

    g = g_ref[0]
    for l in range(1, _NC):
        mask = (g == l).astype(jnp.float32)
        acc = None
        for c in range(_NC):
            t = jnp.maximum(jnp.abs(x_ref[0, c] - a_ref[l, c]) * s_ref[l, c],
                            _DELTA)
            acc = t if acc is None else acc + t
        masked = acc * mask
        sums_ref[l, :] += jnp.sum(masked, axis=0)
        cnts_ref[l, :] += jnp.sum(mask, axis=0)


def _owloss_tc_part(logits, sem_gt, a_tab, s_tab, b_img, row0):
    B, C, H, W = logits.shape
    nh = (H - row0) // _TCBH
    h0 = row0 // _TCBH
    out = pl.pallas_call(
        _tc_tile_body,
        grid=(nh,),
        in_specs=[
            pl.BlockSpec((1, C, _TCBH, W), lambda i: (b_img, 0, h0 + i, 0)),
            pl.BlockSpec((1, _TCBH, W), lambda i: (b_img, h0 + i, 0)),
            pl.BlockSpec(memory_space=pltpu.SMEM),
            pl.BlockSpec(memory_space=pltpu.SMEM),
        ],
        out_specs=[
            pl.BlockSpec((24, W), lambda i: (0, 0)),
            pl.BlockSpec((24, W), lambda i: (0, 0)),
        ],
        out_shape=[
            jax.ShapeDtypeStruct((24, W), jnp.float32),
            jax.ShapeDtypeStruct((24, W), jnp.float32),
        ],
    )(logits, sem_gt, a_tab, s_tab)
    return out


@jax.jit
def _owloss_sc(x4, lab, t_tab):
    mesh = plsc.VectorSubcoreMesh(core_axis_name="c", subcore_axis_name="s")
    nw = mesh.num_cores * mesh.num_subcores
    B, C, H, W = x4.shape
    body = functools.partial(_sc_body, _RPW)
    parts = pl.kernel(
        body,
        out_type=jax.ShapeDtypeStruct((nw, 2, _BINS), jnp.float32),
        mesh=mesh,
        compiler_params=pltpu.CompilerParams(needs_layout_passes=False),
        scratch_types=[
            pltpu.VMEM((t_tab.shape[0],), jnp.int32),
            pltpu.VMEM((2, _TR, _TC), jnp.int32),
            pltpu.VMEM((2, _NC, _TR, _TC), jnp.float32),
            pltpu.VMEM((_BINS,), jnp.float32),
            pltpu.VMEM((_BINS,), jnp.float32),
            pltpu.SemaphoreType.DMA,
            pltpu.SemaphoreType.DMA,
            pltpu.SemaphoreType.DMA,
            pltpu.SemaphoreType.DMA,
        ],
    )(x4, lab, t_tab)
    return parts


def kernel(logits, sem_gt, is_train, previous_features, previous_count, var):
    B, C, H, W = logits.shape
    pos = var > 0
    absv = jnp.abs(var)
    nzmin = jnp.min(jnp.where(pos, absv, jnp.inf), axis=1, keepdims=True)
    variance = jnp.where(pos, nzmin, var)
    norm_var = variance / nzmin
    s_full = 1.0 / (norm_var + _SMOOTH)

    a_u = lax.bitcast_convert_type(
        previous_features.T.astype(jnp.bfloat16), jnp.uint16).astype(jnp.uint32)
    s_u = lax.bitcast_convert_type(
        s_full.T.astype(jnp.bfloat16), jnp.uint16).astype(jnp.uint32)
    word = ((s_u << 16) | a_u).astype(jnp.uint32)
    t_tab = jnp.zeros((_NC, 24), jnp.uint32).at[:, :_NC].set(word)
    t_tab = lax.bitcast_convert_type(t_tab.reshape(-1), jnp.int32)

    lab = sem_gt.astype(jnp.int32)

    parts = _owloss_sc(logits, lab, t_tab)
    tc_sums2d, tc_cnts2d = _owloss_tc_part(
        logits, lab, previous_features, s_full, B - 1, 0)
    tc_sums = jnp.sum(tc_sums2d[: _NC], axis=1)
    tc_cnts = jnp.sum(tc_cnts2d[: _NC], axis=1)
    sums = parts[:, 0, :].reshape(-1, _NC, _L).sum(axis=(0, 2))
    cnts = parts[:, 1, :].reshape(-1, _NC, _L).sum(axis=(0, 2))
    sums = sums + tc_sums
    cnts = cnts + tc_cnts
    sums = sums - (_NC * _DELTA) * cnts

    lbl = jnp.arange(_NC)
    denom = jnp.maximum(cnts * _NC, 1.0)
    mean_val = sums / denom
    cond = (lbl >= 1) & (cnts > 0) & (previous_count > 0) & (jnp.sum(var, axis=1) != 0)
    acc = jnp.sum(jnp.where(cond, mean_val, 0.0))
    return jnp.clip(acc, 0.0, 20.0)
